# baseline (speedup 1.0000x reference)
import functools

import jax
import jax.numpy as jnp
from jax import lax
from jax.experimental import pallas as pl
from jax.experimental.pallas import tpu as pltpu
from jax.experimental.pallas import tpu_sc as plsc

NB = 2048
LANES = 16
NW = 32
ROWS = 4 * 4096
ROWS_PER_W = ROWS // NW
CHUNK_ROWS = 16
NCH = ROWS_PER_W // CHUNK_ROWS
VPC = CHUNK_ROWS * NB // LANES
UNROLL = 8
NACC = 8


def _vreg(buf, g):
    return buf[g >> 7, pl.ds((g & 127) * LANES, LANES)]


def _minmax_call(x2d):
    mesh = plsc.VectorSubcoreMesh(core_axis_name="c", subcore_axis_name="s")

    @functools.partial(
        pl.kernel,
        mesh=mesh,
        compiler_params=pltpu.CompilerParams(needs_layout_passes=False),
        out_type=jax.ShapeDtypeStruct((NW * 2 * LANES,), jnp.float32),
        scratch_types=[
            pltpu.VMEM((CHUNK_ROWS, NB), jnp.float32),
            pltpu.VMEM((CHUNK_ROWS, NB), jnp.float32),
            pltpu.VMEM((2 * LANES,), jnp.float32),
            pltpu.SemaphoreType.DMA,
            pltpu.SemaphoreType.DMA,
        ],
    )
    def minmax_kernel(x_hbm, out_hbm, buf0, buf1, obuf, sem0, sem1):
        wid = lax.axis_index("s") * 2 + lax.axis_index("c")
        base = wid * ROWS_PER_W

        def start(c, buf, sem):
            pltpu.async_copy(
                x_hbm.at[pl.ds(base + c * CHUNK_ROWS, CHUNK_ROWS), :],
                buf, sem)

        def wait(buf, sem):
            pltpu.make_async_copy(
                x_hbm.at[pl.ds(base, CHUNK_ROWS), :], buf, sem).wait()

        def process(buf, accs):
            @plsc.parallel_loop(0, VPC // NACC, carry=accs)
            def _inner(i, a):
                new = []
                for u in range(NACC):
                    v = _vreg(buf, i * NACC + u)
                    mn, mx = a[u]
                    new.append((jnp.minimum(mn, v), jnp.maximum(mx, v)))
                return tuple(new)

            return _inner

        inf = jnp.full((LANES,), jnp.inf, jnp.float32)
        accs0 = tuple((inf, -inf) for _ in range(NACC))

        start(0, buf0, sem0)

        def outer(j, accs):
            start(2 * j + 1, buf1, sem1)
            wait(buf0, sem0)
            accs = process(buf0, accs)

            @pl.when(j < NCH // 2 - 1)
            def _():
                start(2 * j + 2, buf0, sem0)

            wait(buf1, sem1)
            return process(buf1, accs)

        accs = lax.fori_loop(0, NCH // 2, outer, accs0)

        mn, mx = accs[0]
        for u in range(1, NACC):
            mn = jnp.minimum(mn, accs[u][0])
            mx = jnp.maximum(mx, accs[u][1])
        obuf[pl.ds(0, LANES)] = mn
        obuf[pl.ds(LANES, LANES)] = mx
        pltpu.sync_copy(obuf, out_hbm.at[pl.ds(wid * 2 * LANES, 2 * LANES)])

    return minmax_kernel(x2d)


def _hist_call(x2d, params):
    mesh = plsc.VectorSubcoreMesh(core_axis_name="c", subcore_axis_name="s")

    @functools.partial(
        pl.kernel,
        mesh=mesh,
        compiler_params=pltpu.CompilerParams(needs_layout_passes=False),
        out_type=jax.ShapeDtypeStruct((NW * NB,), jnp.float32),
        scratch_types=[
            pltpu.VMEM((CHUNK_ROWS, NB), jnp.float32),
            pltpu.VMEM((CHUNK_ROWS, NB), jnp.float32),
            pltpu.VMEM((LANES * NB,), jnp.float32),
            pltpu.VMEM((NB,), jnp.float32),
            pltpu.VMEM((2 * LANES,), jnp.float32),
            pltpu.SemaphoreType.DMA,
            pltpu.SemaphoreType.DMA,
        ],
    )
    def hist_kernel(x_hbm, p_hbm, out_hbm, buf0, buf1, hist, outbuf, pbuf,
                    sem0, sem1):
        wid = lax.axis_index("s") * 2 + lax.axis_index("c")
        base = wid * ROWS_PER_W

        pltpu.sync_copy(p_hbm, pbuf)
        minv = pbuf[pl.ds(0, LANES)]
        wv = pbuf[pl.ds(LANES, LANES)]
        laneoff = plsc.bitcast(lax.iota(jnp.int32, LANES) * NB, jnp.uint32)
        ones = jnp.ones((LANES,), jnp.float32)
        zeros = jnp.zeros((LANES,), jnp.float32)

        @plsc.parallel_loop(0, LANES * NB // LANES, unroll=8)
        def _zinit(i):
            hist[pl.ds(i * LANES, LANES)] = zeros

        def start(c, buf, sem):
            pltpu.async_copy(
                x_hbm.at[pl.ds(base + c * CHUNK_ROWS, CHUNK_ROWS), :],
                buf, sem)

        def wait(buf, sem):
            pltpu.make_async_copy(
                x_hbm.at[pl.ds(base, CHUNK_ROWS), :], buf, sem).wait()

        def process(buf):
            @plsc.parallel_loop(0, VPC, unroll=UNROLL)
            def _inner(i):
                v = _vreg(buf, i)
                q = (v - minv) / wv
                idx_u = plsc.bitcast(q.astype(jnp.int32), jnp.uint32)
                idx = jnp.minimum(idx_u, jnp.uint32(NB - 1)) + laneoff
                plsc.addupdate_scatter(
                    hist, [plsc.bitcast(idx, jnp.int32)], ones)

        start(0, buf0, sem0)

        def outer(j, carry):
            start(2 * j + 1, buf1, sem1)
            wait(buf0, sem0)
            process(buf0)

            @pl.when(j < NCH // 2 - 1)
            def _():
                start(2 * j + 2, buf0, sem0)

            wait(buf1, sem1)
            process(buf1)
            return carry

        lax.fori_loop(0, NCH // 2, outer, 0)

        @plsc.parallel_loop(0, NB // LANES, unroll=2)
        def _red(g):
            acc = zeros
            for l in range(LANES):
                acc = acc + hist[pl.ds(l * NB + g * LANES, LANES)]
            outbuf[pl.ds(g * LANES, LANES)] = acc

        pltpu.sync_copy(outbuf, out_hbm.at[pl.ds(wid * NB, NB)])

    return hist_kernel(x2d, params)


def kernel(x):
    x2d = x.reshape(ROWS, NB)
    mm = _minmax_call(x2d).reshape(NW, 2, LANES)
    min_val = jnp.min(mm[:, 0, :])
    max_val = jnp.max(mm[:, 1, :])
    bin_width = (max_val - min_val) / NB
    safe_width = jnp.where(bin_width == 0, jnp.float32(1.0), bin_width)
    params = jnp.concatenate(
        [jnp.full((LANES,), min_val), jnp.full((LANES,), safe_width)])
    partials = _hist_call(x2d, params)
    histogram = partials.reshape(NW, NB).sum(0)
    return (x, histogram, min_val, max_val)


# speedup vs baseline: 1.0380x; 1.0380x over previous
import functools

import jax
import jax.numpy as jnp
from jax import lax
from jax.experimental import pallas as pl
from jax.experimental.pallas import tpu as pltpu
from jax.experimental.pallas import tpu_sc as plsc

NB = 2048
LANES = 16
NW = 32
ROWS = 4 * 4096
ROWS_PER_W = ROWS // NW
CHUNK_ROWS = 16
NCH = ROWS_PER_W // CHUNK_ROWS
VPC = CHUNK_ROWS * NB // LANES
UNROLL = 8


def _vreg(buf, g):
    return buf[g >> 7, pl.ds((g & 127) * LANES, LANES)]


def _minmax_body(x_ref, min_ref, max_ref):
    i = pl.program_id(0)
    bmin = jnp.min(x_ref[...])
    bmax = jnp.max(x_ref[...])

    @pl.when(i == 0)
    def _():
        min_ref[0, 0] = bmin
        max_ref[0, 0] = bmax

    @pl.when(i != 0)
    def _():
        min_ref[0, 0] = jnp.minimum(min_ref[0, 0], bmin)
        max_ref[0, 0] = jnp.maximum(max_ref[0, 0], bmax)


def _minmax_call(x2d):
    block_rows = 1024
    return pl.pallas_call(
        _minmax_body,
        grid=(x2d.shape[0] // block_rows,),
        in_specs=[pl.BlockSpec((block_rows, x2d.shape[1]), lambda i: (i, 0))],
        out_specs=[
            pl.BlockSpec(memory_space=pltpu.SMEM),
            pl.BlockSpec(memory_space=pltpu.SMEM),
        ],
        out_shape=[
            jax.ShapeDtypeStruct((1, 1), jnp.float32),
            jax.ShapeDtypeStruct((1, 1), jnp.float32),
        ],
    )(x2d)


def _hist_call(x2d, params):
    mesh = plsc.VectorSubcoreMesh(core_axis_name="c", subcore_axis_name="s")

    @functools.partial(
        pl.kernel,
        mesh=mesh,
        compiler_params=pltpu.CompilerParams(needs_layout_passes=False),
        out_type=jax.ShapeDtypeStruct((NW * NB,), jnp.float32),
        scratch_types=[
            pltpu.VMEM((CHUNK_ROWS, NB), jnp.float32),
            pltpu.VMEM((CHUNK_ROWS, NB), jnp.float32),
            pltpu.VMEM((LANES * NB,), jnp.float32),
            pltpu.VMEM((NB,), jnp.float32),
            pltpu.VMEM((2 * LANES,), jnp.float32),
            pltpu.SemaphoreType.DMA,
            pltpu.SemaphoreType.DMA,
        ],
    )
    def hist_kernel(x_hbm, p_hbm, out_hbm, buf0, buf1, hist, outbuf, pbuf,
                    sem0, sem1):
        wid = lax.axis_index("s") * 2 + lax.axis_index("c")
        base = wid * ROWS_PER_W

        pltpu.sync_copy(p_hbm, pbuf)
        minv = pbuf[pl.ds(0, LANES)]
        wv = pbuf[pl.ds(LANES, LANES)]
        laneoff = plsc.bitcast(lax.iota(jnp.int32, LANES) * NB, jnp.uint32)
        ones = jnp.ones((LANES,), jnp.float32)
        zeros = jnp.zeros((LANES,), jnp.float32)

        @plsc.parallel_loop(0, LANES * NB // LANES, unroll=8)
        def _zinit(i):
            hist[pl.ds(i * LANES, LANES)] = zeros

        def start(c, buf, sem):
            pltpu.async_copy(
                x_hbm.at[pl.ds(base + c * CHUNK_ROWS, CHUNK_ROWS), :],
                buf, sem)

        def wait(buf, sem):
            pltpu.make_async_copy(
                x_hbm.at[pl.ds(base, CHUNK_ROWS), :], buf, sem).wait()

        def process(buf):
            @plsc.parallel_loop(0, VPC, unroll=UNROLL)
            def _inner(i):
                v = _vreg(buf, i)
                q = (v - minv) / wv
                idx_u = plsc.bitcast(q.astype(jnp.int32), jnp.uint32)
                idx = jnp.minimum(idx_u, jnp.uint32(NB - 1)) + laneoff
                plsc.addupdate_scatter(
                    hist, [plsc.bitcast(idx, jnp.int32)], ones)

        start(0, buf0, sem0)

        def outer(j, carry):
            start(2 * j + 1, buf1, sem1)
            wait(buf0, sem0)
            process(buf0)

            @pl.when(j < NCH // 2 - 1)
            def _():
                start(2 * j + 2, buf0, sem0)

            wait(buf1, sem1)
            process(buf1)
            return carry

        lax.fori_loop(0, NCH // 2, outer, 0)

        @plsc.parallel_loop(0, NB // LANES, unroll=2)
        def _red(g):
            acc = zeros
            for l in range(LANES):
                acc = acc + hist[pl.ds(l * NB + g * LANES, LANES)]
            outbuf[pl.ds(g * LANES, LANES)] = acc

        pltpu.sync_copy(outbuf, out_hbm.at[pl.ds(wid * NB, NB)])

    return hist_kernel(x2d, params)


def kernel(x):
    x2d = x.reshape(ROWS, NB)
    mn, mx = _minmax_call(x2d)
    min_val = mn[0, 0]
    max_val = mx[0, 0]
    bin_width = (max_val - min_val) / NB
    safe_width = jnp.where(bin_width == 0, jnp.float32(1.0), bin_width)
    params = jnp.concatenate(
        [jnp.full((LANES,), min_val), jnp.full((LANES,), safe_width)])
    partials = _hist_call(x2d, params)
    histogram = partials.reshape(NW, NB).sum(0)
    return (x, histogram, min_val, max_val)
